# baseline (speedup 1.0000x reference)
import functools

import jax
import jax.numpy as jnp
from jax import lax
from jax.experimental import pallas as pl
from jax.experimental.pallas import tpu as pltpu
from jax.experimental.pallas import tpu_sc as plsc

B, T, D, S = 16, 2048, 768, 8
V = 100
R = V + T
CHUNK = 64
NV = D // 16

MAIN_ROWS = (R // CHUNK) * CHUNK
TAIL_START = R - CHUNK
NCHUNK0 = 17
NCHUNK1 = 16


def _make_sc_kernel():
  mesh = plsc.VectorSubcoreMesh(core_axis_name="c", subcore_axis_name="s")

  @functools.partial(
      pl.kernel,
      out_type=(jax.ShapeDtypeStruct((2, B * S, D), jnp.float32),
                jax.ShapeDtypeStruct((2, B, 8, D), jnp.float32),
                jax.ShapeDtypeStruct((B * S, 128), jnp.float32)),
      mesh=mesh,
      scratch_types=[
          pltpu.VMEM((CHUNK, D), jnp.float32),
          pltpu.VMEM((CHUNK, D), jnp.float32),
          pltpu.VMEM((16, 128), jnp.int32),
          pltpu.VMEM((16, D), jnp.float32),
          pltpu.VMEM((8, 128), jnp.float32),
          pltpu.SemaphoreType.DMA,
          pltpu.SemaphoreType.DMA,
          pltpu.SemaphoreType.DMA,
      ],
      compiler_params=pltpu.CompilerParams(needs_layout_passes=False),
  )
  def sc_kernel(feat_hbm, tail_hbm, mask_hbm, zeros_hbm,
                out_hbm, vis_hbm, rcp_hbm,
                data0, data1, mbuf, acc, vbuf,
                semd0, semd1, semm):
    c = lax.axis_index("c")
    s = lax.axis_index("s")
    data = (data0, data1)
    semd = (semd0, semd1)
    coff = pl.multiple_of(c * (NCHUNK0 * CHUNK), CHUNK)

    mwait = pltpu.async_copy(mask_hbm.at[s], mbuf, semm)
    pltpu.sync_copy(zeros_hbm, acc)
    mwait.wait()

    lane16 = lax.iota(jnp.int32, 16)
    zero_f = jnp.zeros((16,), jnp.float32)
    zero16 = jnp.zeros((16,), jnp.int32)

    def cbody(r, carry):
      cc = list(carry)
      for v in range(8):
        mv = mbuf[r, pl.ds(v * 16, 16)]
        for g in range(S + 1):
          cc[g] = cc[g] + plsc.all_reduce_population_count(mv == g)
      return tuple(cc)

    cnt = plsc.parallel_loop(
        0, 16, 1, carry=tuple(zero16 for _ in range(S + 1)))(cbody)

    run = jnp.full((16,), V, jnp.int32)
    bvec = zero16
    for g in range(S + 1):
      bvec = jnp.where(lane16 == g, run, bvec)
      run = run + cnt[g]
    bvec = jnp.where(lane16 == (S + 1), run, bvec)

    def vext(i):
      return jnp.max(jnp.where(lane16 == i, bvec, 0))

    def accum_ranges(bsel, cstart, clo, chi):
      db = data[bsel]

      def seg_body(q, carry):
        lo = jnp.where(q == 0, 0, vext(q))
        hi = jnp.where(q == 0, V, vext(q + 1))
        row = jnp.where(q == 0, S, q - 1)
        l = jnp.maximum(lo, clo)
        h = jnp.minimum(hi, chi)

        def rowbody(gr, regs):
          lr = gr - cstart
          return tuple(
              regs[i] + db[lr, pl.ds(i * 16, 16)] for i in range(NV))

        regs = plsc.parallel_loop(
            l, h, 1, unroll=2,
            carry=tuple(zero_f for _ in range(NV)))(rowbody)

        @pl.when(h > l)
        def _():
          for i in range(NV):
            plsc.addupdate(acc.at[row, pl.ds(i * 16, 16)], regs[i])
        return carry

      lax.fori_loop(0, S + 1, seg_body, 0)

    def fill_main(k, bsel):
      off = pl.multiple_of(coff + k * CHUNK, CHUNK)
      return pltpu.async_copy(feat_hbm.at[s, pl.ds(off, CHUNK), :],
                              data[bsel], semd[bsel])

    def fill_last(bsel):
      @pl.when(c == 0)
      def _():
        fill_main(NCHUNK1, bsel)
      @pl.when(c != 0)
      def _():
        pltpu.async_copy(tail_hbm.at[s], data[bsel], semd[bsel])

    def wait_fill(bsel):
      pltpu.make_async_copy(tail_hbm.at[s], data[bsel], semd[bsel]).wait()

    fill_main(0, 0)

    def pair(t, carry):
      k0 = t * 2
      cs0 = coff + k0 * CHUNK
      wait_fill(0)
      fill_main(k0 + 1, 1)
      accum_ranges(0, cs0, cs0, cs0 + CHUNK)
      wait_fill(1)

      @pl.when(k0 + 2 < NCHUNK1)
      def _():
        fill_main(k0 + 2, 0)
      @pl.when(k0 + 2 == NCHUNK1)
      def _():
        fill_last(0)
      accum_ranges(1, cs0 + CHUNK, cs0 + CHUNK, cs0 + 2 * CHUNK)
      return carry

    lax.fori_loop(0, NCHUNK1 // 2, pair, 0)

    wait_fill(0)

    @pl.when(c == 0)
    def _():
      c16 = coff + NCHUNK1 * CHUNK
      accum_ranges(0, c16, c16, c16 + CHUNK)
    @pl.when(c != 0)
    def _():
      accum_ranges(0, TAIL_START, MAIN_ROWS, R)

    @pl.when(c == 0)
    def _():
      for g in range(S):
        rv = 1.0 / jnp.maximum(cnt[g + 1].astype(jnp.float32), 1.0)
        for v in range(8):
          vbuf[g, pl.ds(v * 16, 16)] = rv
      pltpu.sync_copy(vbuf, rcp_hbm.at[pl.ds(s * S, S), :])

    pltpu.sync_copy(acc.at[pl.ds(0, S)],
                    out_hbm.at[c, pl.ds(s * S, S), :])
    pltpu.sync_copy(acc.at[pl.ds(8, 8)], vis_hbm.at[c, s])

  return sc_kernel


_sc_kernel = _make_sc_kernel()


def _tc_body(part_ref, vis_ref, rcp_ref, w_ref, b_ref, out_ref):
  p = part_ref[0] + part_ref[1]
  rcol = rcp_ref[:, 0:1]
  means = p * rcol
  vis = vis_ref[0, :, 0, :] + vis_ref[1, :, 0, :]
  ii = lax.broadcasted_iota(jnp.int32, (B * S, B), 0) >> 3
  jj = lax.broadcasted_iota(jnp.int32, (B * S, B), 1)
  rmat = jnp.where(ii == jj, 1.0 / V, 0.0)
  w1 = w_ref[0:D, :]
  w2 = w_ref[D:2 * D, :]
  visw = jnp.dot(vis, w2, preferred_element_type=jnp.float32,
                 precision=lax.Precision.HIGHEST)
  vism = jnp.dot(rmat, visw, preferred_element_type=jnp.float32,
                 precision=lax.Precision.HIGHEST)
  out = jnp.dot(means, w1, preferred_element_type=jnp.float32,
                precision=lax.Precision.HIGHEST)
  out_ref[...] = out + vism + b_ref[...]


def _tc_finish(partials, vis, rcp, W, b):
  b2 = b.reshape(1, D)
  return pl.pallas_call(
      _tc_body,
      out_shape=jax.ShapeDtypeStruct((B * S, D), jnp.float32),
  )(partials, vis, rcp, W, b2)


@jax.jit
def kernel(vision_trace_feat, vision_trace_mask, W, b):
  zeros = jnp.zeros((16, D), jnp.float32)
  mask_i = vision_trace_mask.astype(jnp.int32)
  mask4sc = mask_i.reshape(B, 16, 128)
  tail = vision_trace_feat[:, TAIL_START:, :]
  partials, vis, rcp = _sc_kernel(vision_trace_feat, tail, mask4sc, zeros)
  return _tc_finish(partials, vis, rcp, W, b)


# speedup vs baseline: 1.0634x; 1.0634x over previous
import functools

import jax
import jax.numpy as jnp
from jax import lax
from jax.experimental import pallas as pl
from jax.experimental.pallas import tpu as pltpu
from jax.experimental.pallas import tpu_sc as plsc

B, T, D, S = 16, 2048, 768, 8
V = 100
R = V + T
CHUNK = 64
NV = D // 16

MAIN_ROWS = (R // CHUNK) * CHUNK
TAIL_START = R - CHUNK
NCHUNK0 = 17
NCHUNK1 = 16


def _make_sc_kernel():
  mesh = plsc.VectorSubcoreMesh(core_axis_name="c", subcore_axis_name="s")

  @functools.partial(
      pl.kernel,
      out_type=(jax.ShapeDtypeStruct((2, B * S, D), jnp.float32),
                jax.ShapeDtypeStruct((2, B, 8, D), jnp.float32),
                jax.ShapeDtypeStruct((B * S, 128), jnp.float32)),
      mesh=mesh,
      scratch_types=[
          pltpu.VMEM((CHUNK, D), jnp.float32),
          pltpu.VMEM((CHUNK, D), jnp.float32),
          pltpu.VMEM((16, 128), jnp.int32),
          pltpu.VMEM((16, D), jnp.float32),
          pltpu.VMEM((8, 128), jnp.float32),
          pltpu.SemaphoreType.DMA,
          pltpu.SemaphoreType.DMA,
          pltpu.SemaphoreType.DMA,
      ],
      compiler_params=pltpu.CompilerParams(needs_layout_passes=False),
  )
  def sc_kernel(feat_hbm, tail_hbm, mask_hbm, zeros_hbm,
                out_hbm, vis_hbm, rcp_hbm,
                data0, data1, mbuf, acc, vbuf,
                semd0, semd1, semm):
    c = lax.axis_index("c")
    s = lax.axis_index("s")
    data = (data0, data1)
    semd = (semd0, semd1)
    coff = pl.multiple_of(c * (NCHUNK0 * CHUNK), CHUNK)

    mwait = pltpu.async_copy(mask_hbm.at[s], mbuf, semm)
    pltpu.sync_copy(zeros_hbm, acc)
    mwait.wait()

    lane16 = lax.iota(jnp.int32, 16)
    zero_f = jnp.zeros((16,), jnp.float32)
    zero16 = jnp.zeros((16,), jnp.int32)

    def cbody(r, carry):
      cc = list(carry)
      for v in range(8):
        mv = mbuf[r, pl.ds(v * 16, 16)]
        for g in range(S + 1):
          cc[g] = cc[g] + plsc.all_reduce_population_count(mv == g)
      return tuple(cc)

    cnt = plsc.parallel_loop(
        0, 16, 1, carry=tuple(zero16 for _ in range(S + 1)))(cbody)

    run = jnp.full((16,), V, jnp.int32)
    bvec = zero16
    for g in range(S + 1):
      bvec = jnp.where(lane16 == g, run, bvec)
      run = run + cnt[g]
    bvec = jnp.where(lane16 == (S + 1), run, bvec)

    def vext(i):
      return jnp.max(jnp.where(lane16 == i, bvec, 0))

    def accum_ranges(bsel, cstart, clo, chi):
      db = data[bsel]

      def seg_body(q, carry):
        lo = jnp.where(q == 0, 0, vext(q))
        hi = jnp.where(q == 0, V, vext(q + 1))
        row = jnp.where(q == 0, S, q - 1)
        l = jnp.maximum(lo, clo)
        h = jnp.minimum(hi, chi)

        def rowbody(gr, regs):
          lr = gr - cstart
          return tuple(
              regs[i] + db[lr, pl.ds(i * 16, 16)] for i in range(NV))

        regs = lax.fori_loop(l, h, rowbody, tuple(zero_f for _ in range(NV)))

        @pl.when(h > l)
        def _():
          for i in range(NV):
            plsc.addupdate(acc.at[row, pl.ds(i * 16, 16)], regs[i])
        return carry

      lax.fori_loop(0, S + 1, seg_body, 0)

    def fill_main(k, bsel):
      off = pl.multiple_of(coff + k * CHUNK, CHUNK)
      return pltpu.async_copy(feat_hbm.at[s, pl.ds(off, CHUNK), :],
                              data[bsel], semd[bsel])

    def fill_last(bsel):
      @pl.when(c == 0)
      def _():
        fill_main(NCHUNK1, bsel)
      @pl.when(c != 0)
      def _():
        pltpu.async_copy(tail_hbm.at[s], data[bsel], semd[bsel])

    def wait_fill(bsel):
      pltpu.make_async_copy(tail_hbm.at[s], data[bsel], semd[bsel]).wait()

    fill_main(0, 0)

    def pair(t, carry):
      k0 = t * 2
      cs0 = coff + k0 * CHUNK
      wait_fill(0)
      fill_main(k0 + 1, 1)
      accum_ranges(0, cs0, cs0, cs0 + CHUNK)
      wait_fill(1)

      @pl.when(k0 + 2 < NCHUNK1)
      def _():
        fill_main(k0 + 2, 0)
      @pl.when(k0 + 2 == NCHUNK1)
      def _():
        fill_last(0)
      accum_ranges(1, cs0 + CHUNK, cs0 + CHUNK, cs0 + 2 * CHUNK)
      return carry

    lax.fori_loop(0, NCHUNK1 // 2, pair, 0)

    wait_fill(0)

    @pl.when(c == 0)
    def _():
      c16 = coff + NCHUNK1 * CHUNK
      accum_ranges(0, c16, c16, c16 + CHUNK)
    @pl.when(c != 0)
    def _():
      accum_ranges(0, TAIL_START, MAIN_ROWS, R)

    @pl.when(c == 0)
    def _():
      for g in range(S):
        rv = 1.0 / jnp.maximum(cnt[g + 1].astype(jnp.float32), 1.0)
        for v in range(8):
          vbuf[g, pl.ds(v * 16, 16)] = rv
      pltpu.sync_copy(vbuf, rcp_hbm.at[pl.ds(s * S, S), :])

    pltpu.sync_copy(acc.at[pl.ds(0, S)],
                    out_hbm.at[c, pl.ds(s * S, S), :])
    pltpu.sync_copy(acc.at[pl.ds(8, 8)], vis_hbm.at[c, s])

  return sc_kernel


_sc_kernel = _make_sc_kernel()


def _tc_body(part_ref, vis_ref, rcp_ref, w_ref, b_ref, out_ref):
  p = part_ref[0] + part_ref[1]
  rcol = rcp_ref[:, 0:1]
  means = p * rcol
  vis = vis_ref[0, :, 0, :] + vis_ref[1, :, 0, :]
  ii = lax.broadcasted_iota(jnp.int32, (B * S, B), 0) >> 3
  jj = lax.broadcasted_iota(jnp.int32, (B * S, B), 1)
  rmat = jnp.where(ii == jj, 1.0 / V, 0.0)
  w1 = w_ref[0:D, :]
  w2 = w_ref[D:2 * D, :]
  visw = jnp.dot(vis, w2, preferred_element_type=jnp.float32,
                 precision=lax.Precision.HIGHEST)
  vism = jnp.dot(rmat, visw, preferred_element_type=jnp.float32,
                 precision=lax.Precision.HIGHEST)
  out = jnp.dot(means, w1, preferred_element_type=jnp.float32,
                precision=lax.Precision.HIGHEST)
  out_ref[...] = out + vism + b_ref[...]


def _tc_finish(partials, vis, rcp, W, b):
  b2 = b.reshape(1, D)
  return pl.pallas_call(
      _tc_body,
      out_shape=jax.ShapeDtypeStruct((B * S, D), jnp.float32),
  )(partials, vis, rcp, W, b2)


@jax.jit
def kernel(vision_trace_feat, vision_trace_mask, W, b):
  zeros = jnp.zeros((16, D), jnp.float32)
  mask_i = vision_trace_mask.astype(jnp.int32)
  mask4sc = mask_i.reshape(B, 16, 128)
  tail = vision_trace_feat[:, TAIL_START:, :]
  partials, vis, rcp = _sc_kernel(vision_trace_feat, tail, mask4sc, zeros)
  return _tc_finish(partials, vis, rcp, W, b)
